# baseline (speedup 1.0000x reference)
import functools

import jax
import jax.numpy as jnp
from jax import lax
from jax.experimental import pallas as pl
from jax.experimental.pallas import tpu as pltpu
from jax.experimental.pallas import tpu_sc as plsc

_VOCAB = 100000
_VPAD = 102400
_D = 64
_BATCH = 4096
_SEQ = 200

_NC = 2
_NS = 16
_NW = _NC * _NS
_NBLK = _BATCH // 16
_BPW = _NBLK // _NW


def _proj_body(x_ref, w_ref, b_ref, o_ref):
    x = x_ref[...]
    w = w_ref[...]
    y = (jnp.dot(x, w, preferred_element_type=jnp.float32)
         + b_ref[...]) * (1.0 / _SEQ)
    o_ref[...] = y.reshape(o_ref.shape)


def _project_table(embed_table, head_w, head_b):
    rows_per_blk = 4096
    grid = _VPAD // rows_per_blk
    return pl.pallas_call(
        _proj_body,
        grid=(grid,),
        in_specs=[
            pl.BlockSpec((rows_per_blk, _D), lambda i: (i, 0)),
            pl.BlockSpec((_D, 1), lambda i: (0, 0)),
            pl.BlockSpec((1, 1), lambda i: (0, 0)),
        ],
        out_specs=pl.BlockSpec((rows_per_blk // 128, 128), lambda i: (i, 0)),
        out_shape=jax.ShapeDtypeStruct((_VPAD // 128, 128), jnp.float32),
    )(embed_table, head_w, head_b.reshape(1, 1))


_IDS_PER_W = _BPW * _SEQ * 16


def _sc_body(proj_hbm, ids_hbm, out_hbm, proj_v, ids_v, out_v):
    wid = lax.axis_index("s") * _NC + lax.axis_index("c")
    pltpu.sync_copy(proj_hbm, proj_v)
    pltpu.sync_copy(ids_hbm.at[pl.ds(wid * _IDS_PER_W, _IDS_PER_W)], ids_v)
    lane_off = lax.iota(jnp.int32, 16) * _SEQ
    for j in range(_BPW):
        def body(s, acc, j=j):
            pos = lane_off + (j * 16 * _SEQ + s)
            idx = plsc.load_gather(ids_v, [pos])
            return acc + plsc.load_gather(proj_v, [idx])
        acc = lax.fori_loop(0, _SEQ, body, jnp.zeros((16,), jnp.float32))
        out_v[pl.ds(j * 16, 16)] = acc
    pltpu.sync_copy(out_v, out_hbm.at[pl.ds(wid * _BPW * 16, _BPW * 16)])


def _gather_sum(proj, ids_flat):
    mesh = plsc.VectorSubcoreMesh(core_axis_name="c", subcore_axis_name="s")
    run = functools.partial(
        pl.kernel,
        mesh=mesh,
        compiler_params=pltpu.CompilerParams(needs_layout_passes=False),
        out_type=jax.ShapeDtypeStruct((_BATCH,), jnp.float32),
        scratch_types=[
            pltpu.VMEM((_VPAD,), jnp.float32),
            pltpu.VMEM((_IDS_PER_W,), jnp.int32),
            pltpu.VMEM((_BPW * 16,), jnp.float32),
        ],
    )(_sc_body)
    return run(proj, ids_flat)


def kernel(input_ids, embed_table, head_w, head_b):
    proj = _project_table(embed_table, head_w, head_b).reshape(_VPAD)
    ids_flat = input_ids.astype(jnp.int32).reshape(-1)
    return _gather_sum(proj, ids_flat)


# speedup vs baseline: 1.0606x; 1.0606x over previous
import functools

import jax
import jax.numpy as jnp
from jax import lax
from jax.experimental import pallas as pl
from jax.experimental.pallas import tpu as pltpu
from jax.experimental.pallas import tpu_sc as plsc

_VOCAB = 100000
_VPAD = 102400
_D = 64
_BATCH = 4096
_SEQ = 200

_NC = 2
_NS = 16
_NW = _NC * _NS
_NBLK = _BATCH // 16
_BPW = _NBLK // _NW


def _proj_body(x_ref, w_ref, b_ref, o_ref):
    x = x_ref[...]
    w = w_ref[...]
    y = (jnp.dot(x, w, preferred_element_type=jnp.float32)
         + b_ref[...]) * (1.0 / _SEQ)
    o_ref[...] = y.reshape(o_ref.shape)


def _project_table(embed_table, head_w, head_b):
    rows_per_blk = 4096
    grid = _VPAD // rows_per_blk
    return pl.pallas_call(
        _proj_body,
        grid=(grid,),
        in_specs=[
            pl.BlockSpec((rows_per_blk, _D), lambda i: (i, 0)),
            pl.BlockSpec((_D, 1), lambda i: (0, 0)),
            pl.BlockSpec((1, 1), lambda i: (0, 0)),
        ],
        out_specs=pl.BlockSpec((rows_per_blk // 128, 128), lambda i: (i, 0)),
        out_shape=jax.ShapeDtypeStruct((_VPAD // 128, 128), jnp.float32),
    )(embed_table, head_w, head_b.reshape(1, 1))


_IDS_PER_W = _BPW * _SEQ * 16


_UNROLL = 8


def _sc_body(proj_hbm, ids_hbm, out_hbm, proj_v, ids_v, out_v):
    wid = lax.axis_index("s") * _NC + lax.axis_index("c")
    pltpu.sync_copy(proj_hbm, proj_v)
    pltpu.sync_copy(ids_hbm.at[pl.ds(wid * _IDS_PER_W, _IDS_PER_W)], ids_v)
    lane_off = lax.iota(jnp.int32, 16) * _SEQ
    for j in range(_BPW):
        def body(s0, acc, j=j):
            for u in range(_UNROLL):
                pos = lane_off + (s0 * _UNROLL + (j * 16 * _SEQ + u))
                idx = plsc.load_gather(ids_v, [pos])
                acc = acc + plsc.load_gather(proj_v, [idx >> 7, idx & 127])
            return acc
        acc = lax.fori_loop(0, _SEQ // _UNROLL, body,
                            jnp.zeros((16,), jnp.float32))
        out_v[pl.ds(j * 16, 16)] = acc
    pltpu.sync_copy(out_v, out_hbm.at[pl.ds(wid * _BPW * 16, _BPW * 16)])


def _gather_sum(proj2d, ids_flat):
    mesh = plsc.VectorSubcoreMesh(core_axis_name="c", subcore_axis_name="s")
    run = functools.partial(
        pl.kernel,
        mesh=mesh,
        compiler_params=pltpu.CompilerParams(needs_layout_passes=False),
        out_type=jax.ShapeDtypeStruct((_BATCH,), jnp.float32),
        scratch_types=[
            pltpu.VMEM((_VPAD // 128, 128), jnp.float32),
            pltpu.VMEM((_IDS_PER_W,), jnp.int32),
            pltpu.VMEM((_BPW * 16,), jnp.float32),
        ],
    )(_sc_body)
    return run(proj2d, ids_flat)


def kernel(input_ids, embed_table, head_w, head_b):
    proj2d = _project_table(embed_table, head_w, head_b)
    ids_flat = input_ids.astype(jnp.int32).reshape(-1)
    return _gather_sum(proj2d, ids_flat)


# speedup vs baseline: 1.0639x; 1.0031x over previous
import functools

import jax
import jax.numpy as jnp
from jax import lax
from jax.experimental import pallas as pl
from jax.experimental.pallas import tpu as pltpu
from jax.experimental.pallas import tpu_sc as plsc

_VOCAB = 100000
_VPAD = 102400
_D = 64
_BATCH = 4096
_SEQ = 200

_NC = 2
_NS = 16
_NW = _NC * _NS
_NBLK = _BATCH // 16
_BPW = _NBLK // _NW


def _proj_body(x_ref, w_ref, b_ref, o_ref):
    x = x_ref[...]
    w = w_ref[...]
    y = (jnp.dot(x, w, preferred_element_type=jnp.float32)
         + b_ref[...]) * (1.0 / _SEQ)
    o_ref[...] = y.reshape(o_ref.shape)


def _project_table(embed_table, head_w, head_b):
    rows_per_blk = 4096
    grid = _VPAD // rows_per_blk
    return pl.pallas_call(
        _proj_body,
        grid=(grid,),
        in_specs=[
            pl.BlockSpec((rows_per_blk, _D), lambda i: (i, 0)),
            pl.BlockSpec((_D, 1), lambda i: (0, 0)),
            pl.BlockSpec((1, 1), lambda i: (0, 0)),
        ],
        out_specs=pl.BlockSpec((rows_per_blk // 128, 128), lambda i: (i, 0)),
        out_shape=jax.ShapeDtypeStruct((_VPAD // 128, 128), jnp.float32),
    )(embed_table, head_w, head_b.reshape(1, 1))


_ROWS_PER_W = _BATCH // _NW
_CHUNK = 64
_UNROLL = 8


def _sc_body(proj_hbm, ids_hbm, out_hbm, proj_v, ids_v, out_v):
    wid = lax.axis_index("s") * _NC + lax.axis_index("c")
    pltpu.sync_copy(proj_hbm, proj_v)
    lane = lax.iota(jnp.int32, 16)
    for c in range(_ROWS_PER_W // _CHUNK):
        pltpu.sync_copy(
            ids_hbm.at[pl.ds(wid * _ROWS_PER_W + c * _CHUNK, _CHUNK), :],
            ids_v)
        for j in range(_CHUNK // 16):
            row = lane + (j * 16)
            def body(s0, acc, row=row):
                for u in range(_UNROLL):
                    s = s0 * _UNROLL + u
                    col = jnp.zeros((16,), jnp.int32) + s
                    idx = plsc.load_gather(ids_v, [row, col])
                    acc = acc + plsc.load_gather(
                        proj_v, [idx >> 7, idx & 127])
                return acc
            acc = lax.fori_loop(0, _SEQ // _UNROLL, body,
                                jnp.zeros((16,), jnp.float32))
            out_v[pl.ds((c * (_CHUNK // 16) + j) * 16, 16)] = acc
    pltpu.sync_copy(out_v, out_hbm.at[pl.ds(wid * _ROWS_PER_W, _ROWS_PER_W)])


def _gather_sum(proj2d, ids):
    mesh = plsc.VectorSubcoreMesh(core_axis_name="c", subcore_axis_name="s")
    run = functools.partial(
        pl.kernel,
        mesh=mesh,
        compiler_params=pltpu.CompilerParams(needs_layout_passes=False),
        out_type=jax.ShapeDtypeStruct((_BATCH,), jnp.float32),
        scratch_types=[
            pltpu.VMEM((_VPAD // 128, 128), jnp.float32),
            pltpu.VMEM((_CHUNK, _SEQ), jnp.int32),
            pltpu.VMEM((_ROWS_PER_W,), jnp.float32),
        ],
    )(_sc_body)
    return run(proj2d, ids)


def kernel(input_ids, embed_table, head_w, head_b):
    proj2d = _project_table(embed_table, head_w, head_b)
    return _gather_sum(proj2d, input_ids.astype(jnp.int32))


# speedup vs baseline: 1.1008x; 1.0347x over previous
import functools

import jax
import jax.numpy as jnp
from jax import lax
from jax.experimental import pallas as pl
from jax.experimental.pallas import tpu as pltpu
from jax.experimental.pallas import tpu_sc as plsc

_VOCAB = 100000
_VPAD = 102400
_D = 64
_BATCH = 4096
_SEQ = 200
_SLO = 128
_SHI = _SEQ - _SLO

_NC = 2
_NS = 16
_NW = _NC * _NS

_GRID = 4
_VROWS = _VPAD // _GRID
_BROWS = _BATCH // _GRID


def _tc_body(x_ref, w_ref, b_ref, ids_ref, proj_ref, lo_ref, hi_ref):
    y = (jnp.dot(x_ref[...], w_ref[...], preferred_element_type=jnp.float32)
         + b_ref[...]) * (1.0 / _SEQ)
    proj_ref[...] = y.reshape(proj_ref.shape)
    ids = ids_ref[...]
    lo_ref[...] = ids[:, :_SLO]
    hi_ref[:, :_SHI] = ids[:, _SLO:]


def _tc_stage(embed_table, head_w, head_b, ids):
    return pl.pallas_call(
        _tc_body,
        grid=(_GRID,),
        in_specs=[
            pl.BlockSpec((_VROWS, _D), lambda i: (i, 0)),
            pl.BlockSpec((_D, 1), lambda i: (0, 0)),
            pl.BlockSpec((1, 1), lambda i: (0, 0)),
            pl.BlockSpec((_BROWS, _SEQ), lambda i: (i, 0)),
        ],
        out_specs=[
            pl.BlockSpec((_VROWS // 128, 128), lambda i: (i, 0)),
            pl.BlockSpec((_BROWS, 128), lambda i: (i, 0)),
            pl.BlockSpec((_BROWS, 128), lambda i: (i, 0)),
        ],
        out_shape=[
            jax.ShapeDtypeStruct((_VPAD // 128, 128), jnp.float32),
            jax.ShapeDtypeStruct((_BATCH, 128), jnp.int32),
            jax.ShapeDtypeStruct((_BATCH, 128), jnp.int32),
        ],
    )(embed_table, head_w, head_b.reshape(1, 1), ids)


_ROWS_PER_W = _BATCH // _NW
_CHUNK = 64
_UNROLL = 8


def _sc_body(proj_hbm, lo_hbm, hi_hbm, out_hbm, proj_v, lo_v, hi_v, out_v):
    wid = lax.axis_index("s") * _NC + lax.axis_index("c")
    pltpu.sync_copy(proj_hbm, proj_v)
    lane = lax.iota(jnp.int32, 16)
    for c in range(_ROWS_PER_W // _CHUNK):
        r0 = wid * _ROWS_PER_W + c * _CHUNK
        pltpu.sync_copy(lo_hbm.at[pl.ds(r0, _CHUNK), :], lo_v)
        pltpu.sync_copy(hi_hbm.at[pl.ds(r0, _CHUNK), :], hi_v)
        for j in range(_CHUNK // 16):
            row = lane + (j * 16)

            def step(ids_ref, s, acc, row=row):
                col = jnp.zeros((16,), jnp.int32) + s
                idx = plsc.load_gather(ids_ref, [row, col])
                return acc + plsc.load_gather(proj_v, [idx >> 7, idx & 127])

            def body_lo(s0, acc):
                for u in range(_UNROLL):
                    acc = step(lo_v, s0 * _UNROLL + u, acc)
                return acc

            def body_hi(s0, acc):
                for u in range(_UNROLL):
                    acc = step(hi_v, s0 * _UNROLL + u, acc)
                return acc

            acc = lax.fori_loop(0, _SLO // _UNROLL, body_lo,
                                jnp.zeros((16,), jnp.float32))
            acc = lax.fori_loop(0, _SHI // _UNROLL, body_hi, acc)
            out_v[pl.ds((c * (_CHUNK // 16) + j) * 16, 16)] = acc
    pltpu.sync_copy(out_v, out_hbm.at[pl.ds(wid * _ROWS_PER_W, _ROWS_PER_W)])


def _gather_sum(proj2d, ids_lo, ids_hi):
    mesh = plsc.VectorSubcoreMesh(core_axis_name="c", subcore_axis_name="s")
    run = functools.partial(
        pl.kernel,
        mesh=mesh,
        compiler_params=pltpu.CompilerParams(needs_layout_passes=False),
        out_type=jax.ShapeDtypeStruct((_BATCH,), jnp.float32),
        scratch_types=[
            pltpu.VMEM((_VPAD // 128, 128), jnp.float32),
            pltpu.VMEM((_CHUNK, 128), jnp.int32),
            pltpu.VMEM((_CHUNK, 128), jnp.int32),
            pltpu.VMEM((_ROWS_PER_W,), jnp.float32),
        ],
    )(_sc_body)
    return run(proj2d, ids_lo, ids_hi)


def kernel(input_ids, embed_table, head_w, head_b):
    proj2d, ids_lo, ids_hi = _tc_stage(
        embed_table, head_w, head_b, input_ids.astype(jnp.int32))
    return _gather_sum(proj2d, ids_lo, ids_hi)


# speedup vs baseline: 2.6413x; 2.3995x over previous
import functools

import jax
import jax.numpy as jnp
from jax import lax
from jax.experimental import pallas as pl
from jax.experimental.pallas import tpu as pltpu
from jax.experimental.pallas import tpu_sc as plsc

_VOCAB = 100000
_VPAD = 102400
_D = 64
_BATCH = 4096
_SEQ = 200

_NC = 2
_NS = 16
_NW = _NC * _NS

_GRID = 4
_VLANES = _VPAD // _GRID
_GPB = (_BATCH // 128) // _GRID


def _tc_body(xt_ref, w_ref, b_ref, idst_ref, proj_ref, ids_ref):
    wb = w_ref[...]
    scale = jnp.float32(1.0 / _SEQ)
    bias = b_ref[0, 0] * scale
    for t in range(_VLANES // 1024):
        rows = []
        for gg in range(8):
            g = t * 8 + gg
            blk = xt_ref[:, g * 128:(g + 1) * 128]
            rows.append(jnp.sum(blk * wb, axis=0, keepdims=True) * scale)
        proj_ref[pl.ds(t * 8, 8), :] = jnp.concatenate(rows, axis=0) + bias
    for gg in range(_GPB):
        ids_ref[gg, :, :] = idst_ref[:, gg * 128:(gg + 1) * 128]


def _tc_stage(embed_table, head_w, head_b, input_ids):
    return pl.pallas_call(
        _tc_body,
        grid=(_GRID,),
        in_specs=[
            pl.BlockSpec((_D, _VLANES), lambda i: (0, i)),
            pl.BlockSpec((_D, 1), lambda i: (0, 0)),
            pl.BlockSpec((1, 1), lambda i: (0, 0)),
            pl.BlockSpec((_SEQ, _GPB * 128), lambda i: (0, i)),
        ],
        out_specs=[
            pl.BlockSpec((_VLANES // 128, 128), lambda i: (i, 0)),
            pl.BlockSpec((_GPB, _SEQ, 128), lambda i: (i, 0, 0)),
        ],
        out_shape=[
            jax.ShapeDtypeStruct((_VPAD // 128, 128), jnp.float32),
            jax.ShapeDtypeStruct((_NW, _SEQ, 128), jnp.int32),
        ],
    )(embed_table.T, head_w, head_b.reshape(1, 1),
      input_ids.astype(jnp.int32).T)


_ROWS_PER_W = _BATCH // _NW
_UNROLL = 8


def _sc_body(proj_hbm, ids_hbm, out_hbm, proj_v, ids_v, out_v):
    wid = lax.axis_index("s") * _NC + lax.axis_index("c")
    pltpu.sync_copy(proj_hbm, proj_v)
    pltpu.sync_copy(ids_hbm.at[wid], ids_v)
    lane = lax.iota(jnp.int32, 16)
    for j in range(_ROWS_PER_W // 16):
        row = lane + (j * 16)

        def body(s0, acc, row=row):
            for u in range(_UNROLL):
                s = jnp.zeros((16,), jnp.int32) + (s0 * _UNROLL + u)
                idx = plsc.load_gather(ids_v, [s, row])
                acc = acc + plsc.load_gather(proj_v, [idx >> 7, idx & 127])
            return acc

        acc = lax.fori_loop(0, _SEQ // _UNROLL, body,
                            jnp.zeros((16,), jnp.float32))
        out_v[pl.ds(j * 16, 16)] = acc
    pltpu.sync_copy(out_v, out_hbm.at[pl.ds(wid * _ROWS_PER_W, _ROWS_PER_W)])


def _gather_sum(proj2d, ids_packed):
    mesh = plsc.VectorSubcoreMesh(core_axis_name="c", subcore_axis_name="s")
    run = functools.partial(
        pl.kernel,
        mesh=mesh,
        compiler_params=pltpu.CompilerParams(needs_layout_passes=False),
        out_type=jax.ShapeDtypeStruct((_BATCH,), jnp.float32),
        scratch_types=[
            pltpu.VMEM((_VPAD // 128, 128), jnp.float32),
            pltpu.VMEM((_SEQ, 128), jnp.int32),
            pltpu.VMEM((_ROWS_PER_W,), jnp.float32),
        ],
    )(_sc_body)
    return run(proj2d, ids_packed)


def kernel(input_ids, embed_table, head_w, head_b):
    proj2d, ids_packed = _tc_stage(embed_table, head_w, head_b, input_ids)
    return _gather_sum(proj2d, ids_packed)
